# Initial kernel scaffold; baseline (speedup 1.0000x reference)
import jax
import jax.numpy as jnp
from jax.experimental import pallas as pl


def kernel(x, edge_index, fc0_w, fc0_b, W_w, W_b, bn0_g, bn0_b, bn1_g, bn1_b, bn2_g, bn2_b, cheb0_0, cheb0_1, cheb0_2, cheb1_0, cheb1_1, cheb1_2):
    raise NotImplementedError("write your pallas kernel here")


# speedup vs baseline: 4.5582x; 4.5582x over previous
import functools

import jax
import jax.numpy as jnp
from jax import lax
from jax.experimental import pallas as pl
from jax.experimental.pallas import tpu as pltpu
from jax.experimental.pallas import tpu_sc as plsc

N = 10000
E = 160000
D = 256
H = 256
FH = 128
NP = 10016
CH = 80
EPT = E // 16
EPW = E // 32
RB = 1000
NBLK = N // RB
_PREC = lax.Precision.HIGHEST


def _dot(a, b):
    return jnp.dot(a, b, preferred_element_type=jnp.float32, precision=_PREC)


@functools.lru_cache(maxsize=None)
def _sc_kernels():
    mesh = plsc.VectorSubcoreMesh(core_axis_name="c", subcore_axis_name="s")
    sc_params = pltpu.CompilerParams(needs_layout_passes=False)

    def prep_body(srce, dste, degp, dstp, degloc, sbuf, dbuf, obuf):
        c = lax.axis_index("c")
        s = lax.axis_index("s")
        wid = s * 2 + c
        zero16 = jnp.zeros((16,), jnp.float32)
        ones16 = jnp.ones((16,), jnp.float32)
        lanes = jnp.arange(16, dtype=jnp.int32)

        def zloop(i, carry):
            degloc[pl.ds(i * 16, 16)] = zero16
            return carry

        lax.fori_loop(0, N // 16, zloop, 0)

        base = wid * EPW

        def do_vreg(off, nvalid):
            sv = sbuf[pl.ds(off, 16)]
            dv = dbuf[pl.ds(off, 16)]
            valid = sv != dv
            if nvalid < 16:
                valid = valid & (lanes < nvalid)
            plsc.addupdate_scatter(degloc, [sv], ones16, mask=valid)
            obuf[pl.ds(off, 16)] = jnp.where(
                valid, dv, jnp.full((16,), N, jnp.int32))

        def do_chunk(cb, n):
            pltpu.sync_copy(srce.at[pl.ds(cb, n)], sbuf.at[pl.ds(0, n)])
            pltpu.sync_copy(dste.at[pl.ds(cb, n)], dbuf.at[pl.ds(0, n)])
            for v in range(n // 16):
                do_vreg(v * 16, 16)
            if n % 16:
                do_vreg((n // 16) * 16, n % 16)
            pltpu.sync_copy(obuf.at[pl.ds(0, n)], dstp.at[pl.ds(cb, n)])

        nfull = EPW // CH

        def body(g, carry):
            do_chunk(base + g * CH, CH)
            return carry

        lax.fori_loop(0, nfull, body, 0)
        tail = EPW - nfull * CH
        if tail:
            do_chunk(base + nfull * CH, tail)
        pltpu.sync_copy(degloc, degp.at[wid])

    prep = pl.kernel(
        prep_body,
        out_type=(
            jax.ShapeDtypeStruct((32, N), jnp.float32),
            jax.ShapeDtypeStruct((E,), jnp.int32),
        ),
        mesh=mesh,
        compiler_params=sc_params,
        scratch_types=[
            pltpu.VMEM((N,), jnp.float32),
            pltpu.VMEM((CH,), jnp.int32),
            pltpu.VMEM((CH,), jnp.int32),
            pltpu.VMEM((CH,), jnp.int32),
        ],
    )

    def prop_body(ul, ur, srce, dstp, zz, outl, outr, acc, sbuf, dbuf, rows):
        c = lax.axis_index("c")
        s = lax.axis_index("s")
        pltpu.sync_copy(zz.at[pl.ds(s * 624, 624)],
                        acc.at[pl.ds(s * 624, 624)])

        @pl.when(s == 0)
        def _():
            pltpu.sync_copy(zz.at[pl.ds(9984, NP - 9984)],
                            acc.at[pl.ds(9984, NP - 9984)])

        plsc.subcore_barrier()

        def chunk(g, carry):
            base = s * EPT + g * CH
            pltpu.sync_copy(srce.at[pl.ds(base, CH)], sbuf)
            pltpu.sync_copy(dstp.at[pl.ds(base, CH)], dbuf)

            @pl.when(c == 0)
            def _():
                pltpu.sync_copy(ul.at[sbuf], rows)

            @pl.when(c == 1)
            def _():
                pltpu.sync_copy(ur.at[sbuf], rows)

            pltpu.sync_copy(rows, acc.at[dbuf], add=True)
            return carry

        lax.fori_loop(0, EPT // CH, chunk, 0)
        plsc.subcore_barrier()

        @pl.when(c == 0)
        def _():
            pltpu.sync_copy(acc.at[pl.ds(s * 624, 624)],
                            outl.at[pl.ds(s * 624, 624)])

            @pl.when(s == 0)
            def _():
                pltpu.sync_copy(acc.at[pl.ds(9984, N - 9984)],
                                outl.at[pl.ds(9984, N - 9984)])

        @pl.when(c == 1)
        def _():
            pltpu.sync_copy(acc.at[pl.ds(s * 624, 624)],
                            outr.at[pl.ds(s * 624, 624)])

            @pl.when(s == 0)
            def _():
                pltpu.sync_copy(acc.at[pl.ds(9984, N - 9984)],
                                outr.at[pl.ds(9984, N - 9984)])

    prop = pl.kernel(
        prop_body,
        out_type=(
            jax.ShapeDtypeStruct((N, FH), jnp.float32),
            jax.ShapeDtypeStruct((N, FH), jnp.float32),
        ),
        mesh=mesh,
        compiler_params=sc_params,
        scratch_types=[
            pltpu.VMEM_SHARED((NP, FH), jnp.float32),
            pltpu.VMEM((CH,), jnp.int32),
            pltpu.VMEM((CH,), jnp.int32),
            pltpu.VMEM((CH, FH), jnp.float32),
        ],
    )

    return prep, prop


def _fold_body(cw_ref, w_ref, out_ref):
    out_ref[0] = _dot(cw_ref[0], w_ref[...])


def _fold_weights(cheb_stack, W_w):
    return pl.pallas_call(
        _fold_body,
        grid=(6,),
        in_specs=[
            pl.BlockSpec((1, H, H), lambda i: (i, 0, 0)),
            pl.BlockSpec((H, H), lambda i: (0, 0)),
        ],
        out_specs=pl.BlockSpec((1, H, H), lambda i: (i, 0, 0)),
        out_shape=jax.ShapeDtypeStruct((6, H, H), jnp.float32),
    )(cheb_stack, W_w)


def _stats_update(sums_ref, z):
    i = pl.program_id(0)
    s = jnp.sum(z, axis=0, keepdims=True)
    q = jnp.sum(z * z, axis=0, keepdims=True)
    sq = jnp.concatenate([s, q, jnp.zeros((6, H), jnp.float32)], axis=0)

    @pl.when(i == 0)
    def _():
        sums_ref[...] = sq

    @pl.when(i > 0)
    def _():
        sums_ref[...] = sums_ref[...] + sq


def _ka_body(x_ref, w_ref, b_ref, degp_ref, z_ref, sums_ref, dinv_ref):
    z = _dot(x_ref[...], w_ref[...]) + b_ref[...]
    z_ref[...] = z
    _stats_update(sums_ref, z)
    deg = jnp.sum(degp_ref[...], axis=1, keepdims=True)
    dinv_ref[...] = jnp.where(
        deg > 0.0, lax.rsqrt(jnp.maximum(deg, 1.0)), 0.0)


def _ka(x, fc0_w, fc0_b, degp_t):
    return pl.pallas_call(
        _ka_body,
        grid=(NBLK,),
        in_specs=[
            pl.BlockSpec((RB, D), lambda i: (i, 0)),
            pl.BlockSpec((D, H), lambda i: (0, 0)),
            pl.BlockSpec((1, H), lambda i: (0, 0)),
            pl.BlockSpec((RB, 32), lambda i: (i, 0)),
        ],
        out_specs=[
            pl.BlockSpec((RB, H), lambda i: (i, 0)),
            pl.BlockSpec((8, H), lambda i: (0, 0)),
            pl.BlockSpec((RB, 1), lambda i: (i, 0)),
        ],
        out_shape=[
            jax.ShapeDtypeStruct((N, H), jnp.float32),
            jax.ShapeDtypeStruct((8, H), jnp.float32),
            jax.ShapeDtypeStruct((N, 1), jnp.float32),
        ],
    )(x, fc0_w, fc0_b, degp_t)


def _bn_relu(z_ref, sums_ref, g_ref, b_ref):
    mu = sums_ref[0:1, :] * (1.0 / N)
    var = sums_ref[1:2, :] * (1.0 / N) - mu * mu
    rstd = lax.rsqrt(var + 1e-5)
    return jnp.maximum((z_ref[...] - mu) * rstd * g_ref[...] + b_ref[...], 0.0)


def _kb_body(z_ref, sums_ref, g_ref, b_ref, dinv_ref, h_ref, ul_ref, ur_ref):
    h = _bn_relu(z_ref, sums_ref, g_ref, b_ref)
    h_ref[...] = h
    u = h * dinv_ref[...]
    ul_ref[...] = u[:, :FH]
    ur_ref[...] = u[:, FH:]


def _kb(z, sums, g, b, dinv):
    return pl.pallas_call(
        _kb_body,
        grid=(NBLK,),
        in_specs=[
            pl.BlockSpec((RB, H), lambda i: (i, 0)),
            pl.BlockSpec((8, H), lambda i: (0, 0)),
            pl.BlockSpec((1, H), lambda i: (0, 0)),
            pl.BlockSpec((1, H), lambda i: (0, 0)),
            pl.BlockSpec((RB, 1), lambda i: (i, 0)),
        ],
        out_specs=[
            pl.BlockSpec((RB, H), lambda i: (i, 0)),
            pl.BlockSpec((RB, FH), lambda i: (i, 0)),
            pl.BlockSpec((RB, FH), lambda i: (i, 0)),
        ],
        out_shape=[
            jax.ShapeDtypeStruct((N, H), jnp.float32),
            jax.ShapeDtypeStruct((N, FH), jnp.float32),
            jax.ShapeDtypeStruct((N, FH), jnp.float32),
        ],
    )(z, sums, g, b, dinv)


def _kc_body(h_ref, ql_ref, qr_ref, dinv_ref, wf0_ref, wf1_ref,
             acc_ref, vl_ref, vr_ref):
    q = jnp.concatenate([ql_ref[...], qr_ref[...]], axis=1)
    dinv = dinv_ref[...]
    tx1 = -dinv * q
    acc_ref[...] = _dot(h_ref[...], wf0_ref[0]) + _dot(tx1, wf1_ref[0])
    v = dinv * tx1
    vl_ref[...] = v[:, :FH]
    vr_ref[...] = v[:, FH:]


def _kc(h, ql, qr, dinv, wf, layer):
    i0, i1 = 3 * layer, 3 * layer + 1
    return pl.pallas_call(
        _kc_body,
        grid=(NBLK,),
        in_specs=[
            pl.BlockSpec((RB, H), lambda i: (i, 0)),
            pl.BlockSpec((RB, FH), lambda i: (i, 0)),
            pl.BlockSpec((RB, FH), lambda i: (i, 0)),
            pl.BlockSpec((RB, 1), lambda i: (i, 0)),
            pl.BlockSpec((1, H, H), lambda i: (i0, 0, 0)),
            pl.BlockSpec((1, H, H), lambda i: (i1, 0, 0)),
        ],
        out_specs=[
            pl.BlockSpec((RB, H), lambda i: (i, 0)),
            pl.BlockSpec((RB, FH), lambda i: (i, 0)),
            pl.BlockSpec((RB, FH), lambda i: (i, 0)),
        ],
        out_shape=[
            jax.ShapeDtypeStruct((N, H), jnp.float32),
            jax.ShapeDtypeStruct((N, FH), jnp.float32),
            jax.ShapeDtypeStruct((N, FH), jnp.float32),
        ],
    )(h, ql, qr, dinv, wf, wf)


def _kd_body(accin_ref, ql_ref, qr_ref, h_ref, dinv_ref, wf2_ref, wb_ref,
             z_ref, sums_ref):
    q = jnp.concatenate([ql_ref[...], qr_ref[...]], axis=1)
    tx2 = -2.0 * dinv_ref[...] * q - h_ref[...]
    z = accin_ref[...] + _dot(tx2, wf2_ref[0]) + wb_ref[...]
    z_ref[...] = z
    _stats_update(sums_ref, z)


def _kd(accin, ql, qr, h, dinv, wf, layer, W_b):
    i2 = 3 * layer + 2
    return pl.pallas_call(
        _kd_body,
        grid=(NBLK,),
        in_specs=[
            pl.BlockSpec((RB, H), lambda i: (i, 0)),
            pl.BlockSpec((RB, FH), lambda i: (i, 0)),
            pl.BlockSpec((RB, FH), lambda i: (i, 0)),
            pl.BlockSpec((RB, H), lambda i: (i, 0)),
            pl.BlockSpec((RB, 1), lambda i: (i, 0)),
            pl.BlockSpec((1, H, H), lambda i: (i2, 0, 0)),
            pl.BlockSpec((1, H), lambda i: (0, 0)),
        ],
        out_specs=[
            pl.BlockSpec((RB, H), lambda i: (i, 0)),
            pl.BlockSpec((8, H), lambda i: (0, 0)),
        ],
        out_shape=[
            jax.ShapeDtypeStruct((N, H), jnp.float32),
            jax.ShapeDtypeStruct((8, H), jnp.float32),
        ],
    )(accin, ql, qr, h, dinv, wf, W_b)


def _kh_body(z_ref, sums_ref, g_ref, b_ref, h_ref):
    h_ref[...] = _bn_relu(z_ref, sums_ref, g_ref, b_ref)


def _kh(z, sums, g, b):
    return pl.pallas_call(
        _kh_body,
        grid=(NBLK,),
        in_specs=[
            pl.BlockSpec((RB, H), lambda i: (i, 0)),
            pl.BlockSpec((8, H), lambda i: (0, 0)),
            pl.BlockSpec((1, H), lambda i: (0, 0)),
            pl.BlockSpec((1, H), lambda i: (0, 0)),
        ],
        out_specs=pl.BlockSpec((RB, H), lambda i: (i, 0)),
        out_shape=jax.ShapeDtypeStruct((N, H), jnp.float32),
    )(z, sums, g, b)


def kernel(x, edge_index, fc0_w, fc0_b, W_w, W_b, bn0_g, bn0_b, bn1_g, bn1_b,
           bn2_g, bn2_b, cheb0_0, cheb0_1, cheb0_2, cheb1_0, cheb1_1, cheb1_2):
    prep, prop = _sc_kernels()

    srce = edge_index[0]
    dste = edge_index[1]
    fc0_b2 = fc0_b.reshape(1, H)
    W_b2 = W_b.reshape(1, H)
    bn_g = (bn0_g.reshape(1, H), bn1_g.reshape(1, H), bn2_g.reshape(1, H))
    bn_b = (bn0_b.reshape(1, H), bn1_b.reshape(1, H), bn2_b.reshape(1, H))
    zz = jnp.zeros((NP, FH), jnp.float32)

    cheb_stack = jnp.stack(
        [cheb0_0, cheb0_1, cheb0_2, cheb1_0, cheb1_1, cheb1_2])
    wf = _fold_weights(cheb_stack, W_w)

    degp, dstp = prep(srce, dste)
    degp_t = degp.T

    z0, sums0, dinv = _ka(x, fc0_w, fc0_b2, degp_t)
    h = _kb(z0, sums0, bn_g[0], bn_b[0], dinv)
    h0, ul, ur = h

    for layer in range(2):
        q1l, q1r = prop(ul, ur, srce, dstp, zz)
        acc, vl, vr = _kc(h0, q1l, q1r, dinv, wf, layer)
        q2l, q2r = prop(vl, vr, srce, dstp, zz)
        z, sums = _kd(acc, q2l, q2r, h0, dinv, wf, layer, W_b2)
        if layer == 0:
            h0, ul, ur = _kb(z, sums, bn_g[1], bn_b[1], dinv)
        else:
            return _kh(z, sums, bn_g[2], bn_b[2])


# speedup vs baseline: 8.3313x; 1.8278x over previous
import functools

import jax
import jax.numpy as jnp
from jax import lax
from jax.experimental import pallas as pl
from jax.experimental.pallas import tpu as pltpu
from jax.experimental.pallas import tpu_sc as plsc

N = 10000
E = 160000
D = 256
H = 256
FH = 128
NP = 10016
CH = 80
PCH = 100
PNC = 100
NBUF = 3
EPT = E // 16
EPW = E // 32
RB = 1000
NBLK = N // RB
_PREC = lax.Precision.HIGHEST


def _dot(a, b):
    return jnp.dot(a, b, preferred_element_type=jnp.float32, precision=_PREC)


@functools.lru_cache(maxsize=None)
def _sc_kernels():
    mesh = plsc.VectorSubcoreMesh(core_axis_name="c", subcore_axis_name="s")
    sc_params = pltpu.CompilerParams(needs_layout_passes=False)

    def prep_body(srce, dste, degp, dstp, degloc, sbuf, dbuf, obuf):
        c = lax.axis_index("c")
        s = lax.axis_index("s")
        wid = s * 2 + c
        zero16 = jnp.zeros((16,), jnp.float32)
        ones16 = jnp.ones((16,), jnp.float32)
        lanes = jnp.arange(16, dtype=jnp.int32)

        def zloop(i, carry):
            degloc[pl.ds(i * 16, 16)] = zero16
            return carry

        lax.fori_loop(0, N // 16, zloop, 0)

        base = wid * EPW

        def do_vreg(off, nvalid):
            sv = sbuf[pl.ds(off, 16)]
            dv = dbuf[pl.ds(off, 16)]
            valid = sv != dv
            if nvalid < 16:
                valid = valid & (lanes < nvalid)
            plsc.addupdate_scatter(degloc, [sv], ones16, mask=valid)
            obuf[pl.ds(off, 16)] = jnp.where(
                valid, dv, jnp.full((16,), N, jnp.int32))

        def do_chunk(cb, n):
            pltpu.sync_copy(srce.at[pl.ds(cb, n)], sbuf.at[pl.ds(0, n)])
            pltpu.sync_copy(dste.at[pl.ds(cb, n)], dbuf.at[pl.ds(0, n)])
            for v in range(n // 16):
                do_vreg(v * 16, 16)
            if n % 16:
                do_vreg((n // 16) * 16, n % 16)
            pltpu.sync_copy(obuf.at[pl.ds(0, n)], dstp.at[pl.ds(cb, n)])

        nfull = EPW // CH

        def body(g, carry):
            do_chunk(base + g * CH, CH)
            return carry

        lax.fori_loop(0, nfull, body, 0)
        tail = EPW - nfull * CH
        if tail:
            do_chunk(base + nfull * CH, tail)
        pltpu.sync_copy(degloc, degp.at[wid])

    prep = pl.kernel(
        prep_body,
        out_type=(
            jax.ShapeDtypeStruct((32, N), jnp.float32),
            jax.ShapeDtypeStruct((E,), jnp.int32),
        ),
        mesh=mesh,
        compiler_params=sc_params,
        scratch_types=[
            pltpu.VMEM((N,), jnp.float32),
            pltpu.VMEM((CH,), jnp.int32),
            pltpu.VMEM((CH,), jnp.int32),
            pltpu.VMEM((CH,), jnp.int32),
        ],
    )

    def prop_body(ul, ur, src2, dst2, zz, outl, outr, acc,
                  is0, is1, is2, id0, id1, id2, r0, r1, r2,
                  gi0, gi1, gi2, g0, g1, g2, s0, s1, s2):
        ibs = (is0, is1, is2)
        ibd = (id0, id1, id2)
        rows = (r0, r1, r2)
        isem = (gi0, gi1, gi2)
        gsem = (g0, g1, g2)
        ssem = (s0, s1, s2)
        c = lax.axis_index("c")
        s = lax.axis_index("s")
        pltpu.sync_copy(zz.at[pl.ds(s * 624, 624)],
                        acc.at[pl.ds(s * 624, 624)])

        @pl.when(s == 0)
        def _():
            pltpu.sync_copy(zz.at[pl.ds(9984, NP - 9984)],
                            acc.at[pl.ds(9984, NP - 9984)])

        rbase = s * PNC

        def idx_start(j, b):
            pltpu.async_copy(src2.at[rbase + j], ibs[b], isem[b])
            pltpu.async_copy(dst2.at[rbase + j], ibd[b], isem[b])

        def idx_wait(j, b):
            pltpu.make_async_copy(src2.at[rbase + j], ibs[b], isem[b]).wait()
            pltpu.make_async_copy(dst2.at[rbase + j], ibd[b], isem[b]).wait()

        def gather_start(j, b):
            @pl.when(c == 0)
            def _():
                pltpu.async_copy(ul.at[ibs[b]], rows[b], gsem[b])

            @pl.when(c == 1)
            def _():
                pltpu.async_copy(ur.at[ibs[b]], rows[b], gsem[b])

        def gather_wait(j, b):
            pltpu.make_async_copy(ul.at[ibs[b]], rows[b], gsem[b]).wait()

        def scatter_start(j, b):
            pltpu.async_copy(rows[b], acc.at[ibd[b]], ssem[b], add=True)

        def scatter_wait(j, b):
            pltpu.make_async_copy(rows[b], acc.at[ibd[b]], ssem[b]).wait()

        idx_start(0, 0)
        idx_start(1, 1)
        plsc.subcore_barrier()
        idx_wait(0, 0)
        gather_start(0, 0)

        def step3(q, carry):
            for b in range(NBUF):
                j = q * NBUF + b
                bn = (b + 1) % NBUF
                bp = (b + 2) % NBUF

                @pl.when(j + 2 < PNC)
                def _():
                    idx_start(j + 2, bp)

                gather_wait(j, b)
                scatter_start(j, b)
                idx_wait(j + 1, bn)

                @pl.when(j >= 2)
                def _():
                    scatter_wait(j - 2, bn)

                gather_start(j + 1, bn)
            return carry

        lax.fori_loop(0, (PNC - 1) // NBUF, step3, 0)
        jlast = PNC - 1
        gather_wait(jlast, jlast % NBUF)
        scatter_start(jlast, jlast % NBUF)
        for i in range(PNC - NBUF, PNC):
            scatter_wait(i, i % NBUF)
        plsc.subcore_barrier()

        @pl.when(c == 0)
        def _():
            pltpu.sync_copy(acc.at[pl.ds(s * 624, 624)],
                            outl.at[pl.ds(s * 624, 624)])

            @pl.when(s == 0)
            def _():
                pltpu.sync_copy(acc.at[pl.ds(9984, N - 9984)],
                                outl.at[pl.ds(9984, N - 9984)])

        @pl.when(c == 1)
        def _():
            pltpu.sync_copy(acc.at[pl.ds(s * 624, 624)],
                            outr.at[pl.ds(s * 624, 624)])

            @pl.when(s == 0)
            def _():
                pltpu.sync_copy(acc.at[pl.ds(9984, N - 9984)],
                                outr.at[pl.ds(9984, N - 9984)])

    prop = pl.kernel(
        prop_body,
        out_type=(
            jax.ShapeDtypeStruct((N, FH), jnp.float32),
            jax.ShapeDtypeStruct((N, FH), jnp.float32),
        ),
        mesh=mesh,
        compiler_params=sc_params,
        scratch_types=(
            [pltpu.VMEM_SHARED((NP, FH), jnp.float32)]
            + [pltpu.VMEM((PCH,), jnp.int32)] * (2 * NBUF)
            + [pltpu.VMEM((PCH, FH), jnp.float32)] * NBUF
            + [pltpu.SemaphoreType.DMA] * (3 * NBUF)
        ),
    )

    return prep, prop


def _fold_body(cw_ref, w_ref, out_ref):
    out_ref[0] = _dot(cw_ref[0], w_ref[...])


def _fold_weights(cheb_stack, W_w):
    return pl.pallas_call(
        _fold_body,
        grid=(6,),
        in_specs=[
            pl.BlockSpec((1, H, H), lambda i: (i, 0, 0)),
            pl.BlockSpec((H, H), lambda i: (0, 0)),
        ],
        out_specs=pl.BlockSpec((1, H, H), lambda i: (i, 0, 0)),
        out_shape=jax.ShapeDtypeStruct((6, H, H), jnp.float32),
    )(cheb_stack, W_w)


def _stats_update(sums_ref, z):
    i = pl.program_id(0)
    s = jnp.sum(z, axis=0, keepdims=True)
    q = jnp.sum(z * z, axis=0, keepdims=True)
    sq = jnp.concatenate([s, q, jnp.zeros((6, H), jnp.float32)], axis=0)

    @pl.when(i == 0)
    def _():
        sums_ref[...] = sq

    @pl.when(i > 0)
    def _():
        sums_ref[...] = sums_ref[...] + sq


def _ka_body(x_ref, w_ref, b_ref, degp_ref, z_ref, sums_ref, dinv_ref):
    z = _dot(x_ref[...], w_ref[...]) + b_ref[...]
    z_ref[...] = z
    _stats_update(sums_ref, z)
    deg = jnp.sum(degp_ref[...], axis=1, keepdims=True)
    dinv_ref[...] = jnp.where(
        deg > 0.0, lax.rsqrt(jnp.maximum(deg, 1.0)), 0.0)


def _ka(x, fc0_w, fc0_b, degp_t):
    return pl.pallas_call(
        _ka_body,
        grid=(NBLK,),
        in_specs=[
            pl.BlockSpec((RB, D), lambda i: (i, 0)),
            pl.BlockSpec((D, H), lambda i: (0, 0)),
            pl.BlockSpec((1, H), lambda i: (0, 0)),
            pl.BlockSpec((RB, 32), lambda i: (i, 0)),
        ],
        out_specs=[
            pl.BlockSpec((RB, H), lambda i: (i, 0)),
            pl.BlockSpec((8, H), lambda i: (0, 0)),
            pl.BlockSpec((RB, 1), lambda i: (i, 0)),
        ],
        out_shape=[
            jax.ShapeDtypeStruct((N, H), jnp.float32),
            jax.ShapeDtypeStruct((8, H), jnp.float32),
            jax.ShapeDtypeStruct((N, 1), jnp.float32),
        ],
    )(x, fc0_w, fc0_b, degp_t)


def _bn_relu(z_ref, sums_ref, g_ref, b_ref):
    mu = sums_ref[0:1, :] * (1.0 / N)
    var = sums_ref[1:2, :] * (1.0 / N) - mu * mu
    rstd = lax.rsqrt(var + 1e-5)
    return jnp.maximum((z_ref[...] - mu) * rstd * g_ref[...] + b_ref[...], 0.0)


def _kb_body(z_ref, sums_ref, g_ref, b_ref, dinv_ref, h_ref, ul_ref, ur_ref):
    h = _bn_relu(z_ref, sums_ref, g_ref, b_ref)
    h_ref[...] = h
    u = h * dinv_ref[...]
    ul_ref[...] = u[:, :FH]
    ur_ref[...] = u[:, FH:]


def _kb(z, sums, g, b, dinv):
    return pl.pallas_call(
        _kb_body,
        grid=(NBLK,),
        in_specs=[
            pl.BlockSpec((RB, H), lambda i: (i, 0)),
            pl.BlockSpec((8, H), lambda i: (0, 0)),
            pl.BlockSpec((1, H), lambda i: (0, 0)),
            pl.BlockSpec((1, H), lambda i: (0, 0)),
            pl.BlockSpec((RB, 1), lambda i: (i, 0)),
        ],
        out_specs=[
            pl.BlockSpec((RB, H), lambda i: (i, 0)),
            pl.BlockSpec((RB, FH), lambda i: (i, 0)),
            pl.BlockSpec((RB, FH), lambda i: (i, 0)),
        ],
        out_shape=[
            jax.ShapeDtypeStruct((N, H), jnp.float32),
            jax.ShapeDtypeStruct((N, FH), jnp.float32),
            jax.ShapeDtypeStruct((N, FH), jnp.float32),
        ],
    )(z, sums, g, b, dinv)


def _kc_body(h_ref, ql_ref, qr_ref, dinv_ref, wf0_ref, wf1_ref,
             acc_ref, vl_ref, vr_ref):
    q = jnp.concatenate([ql_ref[...], qr_ref[...]], axis=1)
    dinv = dinv_ref[...]
    tx1 = -dinv * q
    acc_ref[...] = _dot(h_ref[...], wf0_ref[0]) + _dot(tx1, wf1_ref[0])
    v = dinv * tx1
    vl_ref[...] = v[:, :FH]
    vr_ref[...] = v[:, FH:]


def _kc(h, ql, qr, dinv, wf, layer):
    i0, i1 = 3 * layer, 3 * layer + 1
    return pl.pallas_call(
        _kc_body,
        grid=(NBLK,),
        in_specs=[
            pl.BlockSpec((RB, H), lambda i: (i, 0)),
            pl.BlockSpec((RB, FH), lambda i: (i, 0)),
            pl.BlockSpec((RB, FH), lambda i: (i, 0)),
            pl.BlockSpec((RB, 1), lambda i: (i, 0)),
            pl.BlockSpec((1, H, H), lambda i: (i0, 0, 0)),
            pl.BlockSpec((1, H, H), lambda i: (i1, 0, 0)),
        ],
        out_specs=[
            pl.BlockSpec((RB, H), lambda i: (i, 0)),
            pl.BlockSpec((RB, FH), lambda i: (i, 0)),
            pl.BlockSpec((RB, FH), lambda i: (i, 0)),
        ],
        out_shape=[
            jax.ShapeDtypeStruct((N, H), jnp.float32),
            jax.ShapeDtypeStruct((N, FH), jnp.float32),
            jax.ShapeDtypeStruct((N, FH), jnp.float32),
        ],
    )(h, ql, qr, dinv, wf, wf)


def _kd_body(accin_ref, ql_ref, qr_ref, h_ref, dinv_ref, wf2_ref, wb_ref,
             z_ref, sums_ref):
    q = jnp.concatenate([ql_ref[...], qr_ref[...]], axis=1)
    tx2 = -2.0 * dinv_ref[...] * q - h_ref[...]
    z = accin_ref[...] + _dot(tx2, wf2_ref[0]) + wb_ref[...]
    z_ref[...] = z
    _stats_update(sums_ref, z)


def _kd(accin, ql, qr, h, dinv, wf, layer, W_b):
    i2 = 3 * layer + 2
    return pl.pallas_call(
        _kd_body,
        grid=(NBLK,),
        in_specs=[
            pl.BlockSpec((RB, H), lambda i: (i, 0)),
            pl.BlockSpec((RB, FH), lambda i: (i, 0)),
            pl.BlockSpec((RB, FH), lambda i: (i, 0)),
            pl.BlockSpec((RB, H), lambda i: (i, 0)),
            pl.BlockSpec((RB, 1), lambda i: (i, 0)),
            pl.BlockSpec((1, H, H), lambda i: (i2, 0, 0)),
            pl.BlockSpec((1, H), lambda i: (0, 0)),
        ],
        out_specs=[
            pl.BlockSpec((RB, H), lambda i: (i, 0)),
            pl.BlockSpec((8, H), lambda i: (0, 0)),
        ],
        out_shape=[
            jax.ShapeDtypeStruct((N, H), jnp.float32),
            jax.ShapeDtypeStruct((8, H), jnp.float32),
        ],
    )(accin, ql, qr, h, dinv, wf, W_b)


def _kh_body(z_ref, sums_ref, g_ref, b_ref, h_ref):
    h_ref[...] = _bn_relu(z_ref, sums_ref, g_ref, b_ref)


def _kh(z, sums, g, b):
    return pl.pallas_call(
        _kh_body,
        grid=(NBLK,),
        in_specs=[
            pl.BlockSpec((RB, H), lambda i: (i, 0)),
            pl.BlockSpec((8, H), lambda i: (0, 0)),
            pl.BlockSpec((1, H), lambda i: (0, 0)),
            pl.BlockSpec((1, H), lambda i: (0, 0)),
        ],
        out_specs=pl.BlockSpec((RB, H), lambda i: (i, 0)),
        out_shape=jax.ShapeDtypeStruct((N, H), jnp.float32),
    )(z, sums, g, b)


def kernel(x, edge_index, fc0_w, fc0_b, W_w, W_b, bn0_g, bn0_b, bn1_g, bn1_b,
           bn2_g, bn2_b, cheb0_0, cheb0_1, cheb0_2, cheb1_0, cheb1_1, cheb1_2):
    prep, prop = _sc_kernels()

    srce = edge_index[0]
    dste = edge_index[1]
    fc0_b2 = fc0_b.reshape(1, H)
    W_b2 = W_b.reshape(1, H)
    bn_g = (bn0_g.reshape(1, H), bn1_g.reshape(1, H), bn2_g.reshape(1, H))
    bn_b = (bn0_b.reshape(1, H), bn1_b.reshape(1, H), bn2_b.reshape(1, H))
    zz = jnp.zeros((NP, FH), jnp.float32)

    cheb_stack = jnp.stack(
        [cheb0_0, cheb0_1, cheb0_2, cheb1_0, cheb1_1, cheb1_2])
    wf = _fold_weights(cheb_stack, W_w)

    degp, dstp = prep(srce, dste)
    degp_t = degp.T
    src2 = srce.reshape(16 * PNC, PCH)
    dst2 = dstp.reshape(16 * PNC, PCH)

    z0, sums0, dinv = _ka(x, fc0_w, fc0_b2, degp_t)
    h = _kb(z0, sums0, bn_g[0], bn_b[0], dinv)
    h0, ul, ur = h

    for layer in range(2):
        q1l, q1r = prop(ul, ur, src2, dst2, zz)
        acc, vl, vr = _kc(h0, q1l, q1r, dinv, wf, layer)
        q2l, q2r = prop(vl, vr, src2, dst2, zz)
        z, sums = _kd(acc, q2l, q2r, h0, dinv, wf, layer, W_b2)
        if layer == 0:
            h0, ul, ur = _kb(z, sums, bn_g[1], bn_b[1], dinv)
        else:
            return _kh(z, sums, bn_g[2], bn_b[2])
